# baseline (speedup 1.0000x reference)
import functools

import jax
import jax.numpy as jnp
from jax import lax
from jax.experimental import pallas as pl
from jax.experimental.pallas import tpu as pltpu
from jax.experimental.pallas import tpu_sc as plsc

N = 10000
NPAD = 10240
D = 128
HID = 256
HHALF = 128
PROJ = 32
E = 160000
EPAD = 163840
TAU = 0.4
INV_TAU = 1.0 / TAU
PAD_IDX = 10200

NC, NS, L = 2, 16, 16

RB = 1280
NBLK = NPAD // RB

f32 = jnp.float32


def _fw_body(x_ref, deg_ref, u_ref, keep_ref, acc_ref):
    r = pl.program_id(0)

    @pl.when(r == 0)
    def _():
        acc_ref[...] = jnp.zeros_like(acc_ref)

    xb = jnp.abs(x_ref[...])
    dsl = deg_ref[:, pl.ds(r * RB, RB)]
    degv = dsl[0:1, :] + dsl[1:2, :]
    acc_ref[...] += lax.dot_general(
        degv, xb, (((1,), (0,)), ((), ())), preferred_element_type=f32)

    @pl.when(r == NBLK - 1)
    def _():
        w = jnp.log(acc_ref[...] + 1e-8)
        wmax = jnp.max(w)
        s = (wmax - w) / (wmax - jnp.mean(w) + 1e-8)
        wp = s / (jnp.mean(s) + 1e-8) * 0.1
        wp = jnp.where(wp < 0.7, wp, 0.7)
        keep_ref[...] = jnp.where(u_ref[...] < wp, 0.0, 1.0)


def _tc_keep_mask(xp, deg2, u1):
    return pl.pallas_call(
        _fw_body,
        grid=(NBLK,),
        in_specs=[
            pl.BlockSpec((RB, D), lambda r: (r, 0)),
            pl.BlockSpec((2, NPAD), lambda r: (0, 0)),
            pl.BlockSpec((1, D), lambda r: (0, 0)),
        ],
        out_specs=pl.BlockSpec((1, D), lambda r: (0, 0)),
        out_shape=jax.ShapeDtypeStruct((1, D), f32),
        scratch_shapes=[pltpu.VMEM((1, D), f32)],
    )(xp, deg2, u1)


def _mm1_body(x_ref, keep_ref, w1_ref, o1lo, o1hi, o2lo, o2hi):
    xb = x_ref[...]
    w1 = w1_ref[...]
    h2 = jnp.dot(xb, w1, preferred_element_type=f32)
    h1 = jnp.dot(xb * keep_ref[...], w1, preferred_element_type=f32)
    o1lo[...] = h1[:, :HHALF]
    o1hi[...] = h1[:, HHALF:]
    o2lo[...] = h2[:, :HHALF]
    o2hi[...] = h2[:, HHALF:]


def _tc_mm1(xp, keep, W1):
    half = jax.ShapeDtypeStruct((NPAD, HHALF), f32)
    return pl.pallas_call(
        _mm1_body,
        grid=(NBLK,),
        in_specs=[
            pl.BlockSpec((RB, D), lambda r: (r, 0)),
            pl.BlockSpec((1, D), lambda r: (0, 0)),
            pl.BlockSpec((D, HID), lambda r: (0, 0)),
        ],
        out_specs=[pl.BlockSpec((RB, HHALF), lambda r: (r, 0))] * 4,
        out_shape=[half] * 4,
    )(xp, keep, W1)


def _mm2_body(glo_ref, ghi_ref, a_ref, w2_ref, olo, ohi):
    g = jnp.concatenate([glo_ref[...], ghi_ref[...]], axis=1)
    a = a_ref[0, 0]
    g = jnp.where(g >= 0, g, a * g)
    h = jnp.dot(g, w2_ref[...], preferred_element_type=f32)
    olo[...] = h[:, :HHALF]
    ohi[...] = h[:, HHALF:]


def _tc_mm2(glo, ghi, a1, W2):
    half = jax.ShapeDtypeStruct((NPAD, HHALF), f32)
    return pl.pallas_call(
        _mm2_body,
        grid=(NBLK,),
        in_specs=[
            pl.BlockSpec((RB, HHALF), lambda r: (r, 0)),
            pl.BlockSpec((RB, HHALF), lambda r: (r, 0)),
            pl.BlockSpec((1, 1), lambda r: (0, 0)),
            pl.BlockSpec((HID, HID), lambda r: (0, 0)),
        ],
        out_specs=[pl.BlockSpec((RB, HHALF), lambda r: (r, 0))] * 2,
        out_shape=[half] * 2,
    )(glo, ghi, a1, W2)


def _proj_body(zlo_ref, zhi_ref, a_ref, fw1_ref, fb1_ref, fw2_ref, fb2_ref,
               n_ref):
    z = jnp.concatenate([zlo_ref[...], zhi_ref[...]], axis=1)
    a = a_ref[0, 0]
    z = jnp.where(z >= 0, z, a * z)
    t = jnp.dot(z, fw1_ref[...], preferred_element_type=f32) + fb1_ref[...]
    t = jnp.where(t > 0, t, jnp.exp(t) - 1.0)
    h = jnp.dot(t, fw2_ref[...], preferred_element_type=f32) + fb2_ref[...]
    nrm = jnp.sqrt(jnp.sum(h * h, axis=1, keepdims=True))
    n_ref[...] = h / (nrm + 1e-8)


def _tc_proj(zlo, zhi, a2, fc1_w, fc1_b, fc2_w, fc2_b):
    return pl.pallas_call(
        _proj_body,
        grid=(NBLK,),
        in_specs=[
            pl.BlockSpec((RB, HHALF), lambda r: (r, 0)),
            pl.BlockSpec((RB, HHALF), lambda r: (r, 0)),
            pl.BlockSpec((1, 1), lambda r: (0, 0)),
            pl.BlockSpec((HID, PROJ), lambda r: (0, 0)),
            pl.BlockSpec((1, PROJ), lambda r: (0, 0)),
            pl.BlockSpec((PROJ, HID), lambda r: (0, 0)),
            pl.BlockSpec((1, HID), lambda r: (0, 0)),
        ],
        out_specs=pl.BlockSpec((RB, HID), lambda r: (r, 0)),
        out_shape=jax.ShapeDtypeStruct((NPAD, HID), f32),
    )(zlo, zhi, a2, fc1_w, fc1_b, fc2_w, fc2_b)


bf16 = jnp.bfloat16


def _diag_body(n1_ref, n2_ref, n1b_ref, n2b_ref, d11_ref, d22_ref, d12_ref):
    r = pl.program_id(0)
    a = n1_ref[...]
    b = n2_ref[...]
    n1b_ref[...] = a.astype(bf16)
    n2b_ref[...] = b.astype(bf16)
    sl = pl.ds(r * RB, RB)
    d11_ref[:, sl] = jnp.sum(a * a, axis=1)[None, :]
    d22_ref[:, sl] = jnp.sum(b * b, axis=1)[None, :]
    d12_ref[:, sl] = jnp.sum(a * b, axis=1)[None, :]


def _tc_diag(n1, n2):
    vec = jax.ShapeDtypeStruct((1, NPAD), f32)
    nb = jax.ShapeDtypeStruct((NPAD, HID), bf16)
    return pl.pallas_call(
        _diag_body,
        grid=(NBLK,),
        in_specs=[pl.BlockSpec((RB, HID), lambda r: (r, 0))] * 2,
        out_specs=[pl.BlockSpec((RB, HID), lambda r: (r, 0))] * 2 + [
            pl.BlockSpec((1, NPAD), lambda r: (0, 0))] * 3,
        out_shape=[nb, nb, vec, vec, vec],
    )(n1, n2)


BL = 512
NLB = NPAD // BL


def _loss_body(n1i_ref, n2i_ref, n1j_ref, n2j_ref,
               d11_ref, d22_ref, d12_ref, out_ref,
               r11, r12, r22, c12):
    i = pl.program_id(0)
    j = pl.program_id(1)
    sli = pl.ds(i * BL, BL)
    slj = pl.ds(j * BL, BL)

    @pl.when((i == 0) & (j == 0))
    def _():
        r11[...] = jnp.zeros_like(r11)
        r22[...] = jnp.zeros_like(r22)
        r12[...] = jnp.zeros_like(r12)
        c12[...] = jnp.zeros_like(c12)

    @pl.when(j >= i)
    def _():
        a1 = n1i_ref[...]
        a2 = n2i_ref[...]
        b1 = n1j_ref[...]
        b2 = n2j_ref[...]
        dims = (((1,), (1,)), ((), ()))
        s11 = lax.dot_general(a1, b1, dims, preferred_element_type=f32)
        s22 = lax.dot_general(a2, b2, dims, preferred_element_type=f32)
        s12 = lax.dot_general(a1, b2, dims, preferred_element_type=f32)

        imask = (i * BL + lax.broadcasted_iota(jnp.int32, (BL, 1), 0)) < N
        jmask = (j * BL + lax.broadcasted_iota(jnp.int32, (1, BL), 1)) < N
        m = imask & jmask
        e11 = jnp.where(m, jnp.exp(s11 * INV_TAU), 0.0)
        e22 = jnp.where(m, jnp.exp(s22 * INV_TAU), 0.0)
        e12 = jnp.where(m, jnp.exp(s12 * INV_TAU), 0.0)

        cs11 = jnp.sum(e11, axis=0, keepdims=True)
        cs22 = jnp.sum(e22, axis=0, keepdims=True)
        cs12 = jnp.sum(e12, axis=0, keepdims=True)
        rs12 = jnp.sum(e12, axis=1, keepdims=False)[None, :]

        r11[:, slj] += cs11
        r22[:, slj] += cs22
        c12[:, slj] += cs12
        r12[:, sli] += rs12

        @pl.when(j > i)
        def _():
            s21 = lax.dot_general(a2, b1, dims, preferred_element_type=f32)
            e21 = jnp.where(m, jnp.exp(s21 * INV_TAU), 0.0)
            cs21 = jnp.sum(e21, axis=0, keepdims=True)
            rs21 = jnp.sum(e21, axis=1, keepdims=False)[None, :]
            rs11 = jnp.sum(e11, axis=1, keepdims=False)[None, :]
            rs22 = jnp.sum(e22, axis=1, keepdims=False)[None, :]
            r11[:, sli] += rs11
            r22[:, sli] += rs22
            r12[:, slj] += cs21
            c12[:, sli] += rs21

    @pl.when((i == NLB - 1) & (j == NLB - 1))
    def _():
        mask = lax.broadcasted_iota(jnp.int32, (1, NPAD), 1) < N
        ed11 = jnp.exp(d11_ref[...] * INV_TAU)
        ed22 = jnp.exp(d22_ref[...] * INV_TAU)
        ld12 = d12_ref[...] * INV_TAU
        l1 = -(ld12 - jnp.log(r11[...] - ed11 + r12[...]))
        l2 = -(ld12 - jnp.log(r22[...] - ed22 + c12[...]))
        tot = jnp.sum(jnp.where(mask, 0.5 * (l1 + l2), 0.0))
        out_ref[...] = (tot / N).reshape(1, 1)


def _tc_loss(n1b, n2b, d11, d22, d12):
    vec = pltpu.VMEM((1, NPAD), f32)
    return pl.pallas_call(
        _loss_body,
        grid=(NLB, NLB),
        in_specs=[
            pl.BlockSpec((BL, HID), lambda i, j: (i, 0)),
            pl.BlockSpec((BL, HID), lambda i, j: (i, 0)),
            pl.BlockSpec((BL, HID), lambda i, j: (j, 0)),
            pl.BlockSpec((BL, HID), lambda i, j: (j, 0)),
            pl.BlockSpec((1, NPAD), lambda i, j: (0, 0)),
            pl.BlockSpec((1, NPAD), lambda i, j: (0, 0)),
            pl.BlockSpec((1, NPAD), lambda i, j: (0, 0)),
        ],
        out_specs=pl.BlockSpec((1, 1), lambda i, j: (0, 0)),
        out_shape=jax.ShapeDtypeStruct((1, 1), f32),
        scratch_shapes=[vec] * 4,
    )(n1b, n2b, n1b, n2b, d11, d22, d12)


_EPT = EPAD // (NC * NS)
_DCH = NPAD // NS


def _sc_deg(src_p):
    mesh = plsc.VectorSubcoreMesh(core_axis_name="c", subcore_axis_name="s")

    @functools.partial(
        pl.kernel,
        out_type=jax.ShapeDtypeStruct((NC, NPAD), f32),
        mesh=mesh,
        scratch_types=[
            pltpu.VMEM((_EPT,), jnp.int32),
            pltpu.VMEM((NPAD,), f32),
            pltpu.VMEM((NS, _DCH), f32),
            pltpu.VMEM_SHARED((NS, NPAD), f32),
        ],
        compiler_params=pltpu.CompilerParams(needs_layout_passes=False),
    )
    def k(src_hbm, out_hbm, idx_v, acc_v, red_v, shared):
        cid = lax.axis_index("c")
        sid = lax.axis_index("s")
        wid = cid * NS + sid

        zeros16 = jnp.zeros((L,), f32)

        @pl.loop(0, NPAD // L)
        def _(t):
            acc_v[pl.ds(t * L, L)] = zeros16

        pltpu.sync_copy(src_hbm.at[pl.ds(wid * _EPT, _EPT)], idx_v)
        ones16 = jnp.ones((L,), f32)

        @pl.loop(0, _EPT // L)
        def _(t):
            idx = idx_v[pl.ds(t * L, L)]
            plsc.addupdate_scatter(acc_v, [idx], ones16)

        pltpu.sync_copy(acc_v, shared.at[sid])
        plsc.subcore_barrier()
        pltpu.sync_copy(shared.at[:, pl.ds(sid * _DCH, _DCH)], red_v)

        @pl.loop(0, _DCH // L)
        def _(t):
            v = red_v[0, pl.ds(t * L, L)]
            for r in range(1, NS):
                v = v + red_v[r, pl.ds(t * L, L)]
            acc_v[pl.ds(t * L, L)] = v

        pltpu.sync_copy(acc_v.at[pl.ds(0, _DCH)],
                        out_hbm.at[cid, pl.ds(sid * _DCH, _DCH)])

    return k(src_p)


_SCH = 128
_NCH = EPAD // NS // _SCH
_IB = _NCH // 2
_STR = NPAD // NS


def _sc_segsum(hlo, hhi, src3, dst3):
    mesh = plsc.VectorSubcoreMesh(core_axis_name="c", subcore_axis_name="s")
    half = jax.ShapeDtypeStruct((NPAD, HHALF), f32)

    @functools.partial(
        pl.kernel,
        out_type=(half, half),
        mesh=mesh,
        scratch_types=[
            pltpu.VMEM((_IB, _SCH), jnp.int32),
            pltpu.VMEM((_IB, _SCH), jnp.int32),
            pltpu.VMEM((_SCH, HHALF), f32),
            pltpu.VMEM((_SCH, HHALF), f32),
            pltpu.VMEM_SHARED((NPAD, HHALF), f32),
            pltpu.SemaphoreType.DMA,
            pltpu.SemaphoreType.DMA,
        ],
        compiler_params=pltpu.CompilerParams(needs_layout_passes=False),
    )
    def k(hlo_hbm, hhi_hbm, src_hbm, dst_hbm, olo_hbm, ohi_hbm,
          src_v, dst_v, buf0, buf1, acc, sem0, sem1):
        cid = lax.axis_index("c")
        sid = lax.axis_index("s")
        base = sid * _STR

        zeros16 = jnp.zeros((L,), f32)

        @pl.loop(0, _SCH)
        def _(r):
            for cc in range(HHALF // L):
                buf0[r, pl.ds(cc * L, L)] = zeros16

        for off in range(0, _STR, _SCH):
            pltpu.sync_copy(buf0, acc.at[pl.ds(base + off, _SCH)])
        plsc.subcore_barrier()

        def run_half(h_hbm, o_hbm):
            for blk in range(2):
                pltpu.sync_copy(
                    src_hbm.at[sid, pl.ds(blk * _IB, _IB)], src_v)
                pltpu.sync_copy(
                    dst_hbm.at[sid, pl.ds(blk * _IB, _IB)], dst_v)

                pltpu.async_copy(h_hbm.at[src_v.at[0]], buf0, sem0)
                pltpu.async_copy(h_hbm.at[src_v.at[1]], buf1, sem1)

                @pl.loop(0, _IB, step=2)
                def _(ch):
                    pltpu.make_async_copy(h_hbm.at[src_v.at[ch]], buf0,
                                          sem0).wait()
                    pltpu.sync_copy(buf0, acc.at[dst_v.at[ch]], add=True)

                    @pl.when(ch + 2 < _IB)
                    def _():
                        pltpu.async_copy(h_hbm.at[src_v.at[ch + 2]],
                                         buf0, sem0)

                    pltpu.make_async_copy(h_hbm.at[src_v.at[ch + 1]], buf1,
                                          sem1).wait()
                    pltpu.sync_copy(buf1, acc.at[dst_v.at[ch + 1]], add=True)

                    @pl.when(ch + 3 < _IB)
                    def _():
                        pltpu.async_copy(h_hbm.at[src_v.at[ch + 3]],
                                         buf1, sem1)

            plsc.subcore_barrier()
            pltpu.sync_copy(acc.at[pl.ds(base, _STR)],
                            o_hbm.at[pl.ds(base, _STR)])

        @pl.when(cid == 0)
        def _():
            run_half(hlo_hbm, olo_hbm)

        @pl.when(cid == 1)
        def _():
            run_half(hhi_hbm, ohi_hbm)

    return k(hlo, hhi, src3, dst3)


def kernel(x, edge_index_1, edge_index_2, W1, a1, W2, a2,
           fc1_w, fc1_b, fc2_w, fc2_b):
    xp = jnp.zeros((NPAD, D), f32).at[:N].set(x)
    u1 = jax.random.uniform(jax.random.key(7), (1, D), dtype=f32)

    def prep(ei):
        ei = ei.astype(jnp.int32)
        pad = jnp.full((EPAD - E,), PAD_IDX, jnp.int32)
        shape = (NS, _NCH, _SCH)
        src = jnp.concatenate([ei[1], pad]).reshape(shape)
        dst = jnp.concatenate([ei[0], pad]).reshape(shape)
        return src, dst

    src1, dst1 = prep(edge_index_1)
    src2, dst2 = prep(edge_index_2)

    deg2 = _sc_deg(src1.reshape(EPAD))
    keep = _tc_keep_mask(xp, deg2, u1)
    h1lo, h1hi, h2lo, h2hi = _tc_mm1(xp, keep, W1)

    g1lo, g1hi = _sc_segsum(h1lo, h1hi, src1, dst1)
    g2lo, g2hi = _sc_segsum(h2lo, h2hi, src2, dst2)

    a1r = a1.reshape(1, 1)
    t1lo, t1hi = _tc_mm2(g1lo, g1hi, a1r, W2)
    t2lo, t2hi = _tc_mm2(g2lo, g2hi, a1r, W2)

    z1lo, z1hi = _sc_segsum(t1lo, t1hi, src1, dst1)
    z2lo, z2hi = _sc_segsum(t2lo, t2hi, src2, dst2)

    a2r = a2.reshape(1, 1)
    n1 = _tc_proj(z1lo, z1hi, a2r, fc1_w, fc1_b.reshape(1, PROJ),
                  fc2_w, fc2_b.reshape(1, HID))
    n2 = _tc_proj(z2lo, z2hi, a2r, fc1_w, fc1_b.reshape(1, PROJ),
                  fc2_w, fc2_b.reshape(1, HID))

    n1b, n2b, d11, d22, d12 = _tc_diag(n1, n2)
    return _tc_loss(n1b, n2b, d11, d22, d12)[0, 0]


# speedup vs baseline: 1.1520x; 1.1520x over previous
import functools

import jax
import jax.numpy as jnp
from jax import lax
from jax.experimental import pallas as pl
from jax.experimental.pallas import tpu as pltpu
from jax.experimental.pallas import tpu_sc as plsc

N = 10000
NPAD = 10240
D = 128
HID = 256
HHALF = 128
PROJ = 32
E = 160000
EPAD = 163840
TAU = 0.4
INV_TAU = 1.0 / TAU
PAD_IDX = 10200

NC, NS, L = 2, 16, 16

RB = 1280
NBLK = NPAD // RB

f32 = jnp.float32


def _fw_body(x_ref, deg_ref, u_ref, keep_ref, acc_ref):
    r = pl.program_id(0)

    @pl.when(r == 0)
    def _():
        acc_ref[...] = jnp.zeros_like(acc_ref)

    xb = jnp.abs(x_ref[...])
    dsl = deg_ref[:, pl.ds(r * RB, RB)]
    degv = dsl[0:1, :] + dsl[1:2, :]
    acc_ref[...] += lax.dot_general(
        degv, xb, (((1,), (0,)), ((), ())), preferred_element_type=f32)

    @pl.when(r == NBLK - 1)
    def _():
        w = jnp.log(acc_ref[...] + 1e-8)
        wmax = jnp.max(w)
        s = (wmax - w) / (wmax - jnp.mean(w) + 1e-8)
        wp = s / (jnp.mean(s) + 1e-8) * 0.1
        wp = jnp.where(wp < 0.7, wp, 0.7)
        keep_ref[...] = jnp.where(u_ref[...] < wp, 0.0, 1.0)


def _tc_keep_mask(xp, deg2, u1):
    return pl.pallas_call(
        _fw_body,
        grid=(NBLK,),
        in_specs=[
            pl.BlockSpec((RB, D), lambda r: (r, 0)),
            pl.BlockSpec((2, NPAD), lambda r: (0, 0)),
            pl.BlockSpec((1, D), lambda r: (0, 0)),
        ],
        out_specs=pl.BlockSpec((1, D), lambda r: (0, 0)),
        out_shape=jax.ShapeDtypeStruct((1, D), f32),
        scratch_shapes=[pltpu.VMEM((1, D), f32)],
    )(xp, deg2, u1)


def _mm1_body(x_ref, keep_ref, w1_ref, o1lo, o1hi, o2lo, o2hi):
    xb = x_ref[...]
    w1 = w1_ref[...]
    h2 = jnp.dot(xb, w1, preferred_element_type=f32)
    h1 = jnp.dot(xb * keep_ref[...], w1, preferred_element_type=f32)
    o1lo[...] = h1[:, :HHALF]
    o1hi[...] = h1[:, HHALF:]
    o2lo[...] = h2[:, :HHALF]
    o2hi[...] = h2[:, HHALF:]


def _tc_mm1(xp, keep, W1):
    half = jax.ShapeDtypeStruct((NPAD, HHALF), f32)
    return pl.pallas_call(
        _mm1_body,
        grid=(NBLK,),
        in_specs=[
            pl.BlockSpec((RB, D), lambda r: (r, 0)),
            pl.BlockSpec((1, D), lambda r: (0, 0)),
            pl.BlockSpec((D, HID), lambda r: (0, 0)),
        ],
        out_specs=[pl.BlockSpec((RB, HHALF), lambda r: (r, 0))] * 4,
        out_shape=[half] * 4,
    )(xp, keep, W1)


def _mm2_body(glo_ref, ghi_ref, a_ref, w2_ref, olo, ohi):
    g = jnp.concatenate([glo_ref[...], ghi_ref[...]], axis=1)
    a = a_ref[0, 0]
    g = jnp.where(g >= 0, g, a * g)
    h = jnp.dot(g, w2_ref[...], preferred_element_type=f32)
    olo[...] = h[:, :HHALF]
    ohi[...] = h[:, HHALF:]


def _tc_mm2(glo, ghi, a1, W2):
    half = jax.ShapeDtypeStruct((NPAD, HHALF), f32)
    return pl.pallas_call(
        _mm2_body,
        grid=(NBLK,),
        in_specs=[
            pl.BlockSpec((RB, HHALF), lambda r: (r, 0)),
            pl.BlockSpec((RB, HHALF), lambda r: (r, 0)),
            pl.BlockSpec((1, 1), lambda r: (0, 0)),
            pl.BlockSpec((HID, HID), lambda r: (0, 0)),
        ],
        out_specs=[pl.BlockSpec((RB, HHALF), lambda r: (r, 0))] * 2,
        out_shape=[half] * 2,
    )(glo, ghi, a1, W2)


def _proj_body(zlo_ref, zhi_ref, a_ref, fw1_ref, fb1_ref, fw2_ref, fb2_ref,
               n_ref):
    z = jnp.concatenate([zlo_ref[...], zhi_ref[...]], axis=1)
    a = a_ref[0, 0]
    z = jnp.where(z >= 0, z, a * z)
    t = jnp.dot(z, fw1_ref[...], preferred_element_type=f32) + fb1_ref[...]
    t = jnp.where(t > 0, t, jnp.exp(t) - 1.0)
    h = jnp.dot(t, fw2_ref[...], preferred_element_type=f32) + fb2_ref[...]
    nrm = jnp.sqrt(jnp.sum(h * h, axis=1, keepdims=True))
    n_ref[...] = h / (nrm + 1e-8)


def _tc_proj(zlo, zhi, a2, fc1_w, fc1_b, fc2_w, fc2_b):
    return pl.pallas_call(
        _proj_body,
        grid=(NBLK,),
        in_specs=[
            pl.BlockSpec((RB, HHALF), lambda r: (r, 0)),
            pl.BlockSpec((RB, HHALF), lambda r: (r, 0)),
            pl.BlockSpec((1, 1), lambda r: (0, 0)),
            pl.BlockSpec((HID, PROJ), lambda r: (0, 0)),
            pl.BlockSpec((1, PROJ), lambda r: (0, 0)),
            pl.BlockSpec((PROJ, HID), lambda r: (0, 0)),
            pl.BlockSpec((1, HID), lambda r: (0, 0)),
        ],
        out_specs=pl.BlockSpec((RB, HID), lambda r: (r, 0)),
        out_shape=jax.ShapeDtypeStruct((NPAD, HID), f32),
    )(zlo, zhi, a2, fc1_w, fc1_b, fc2_w, fc2_b)


bf16 = jnp.bfloat16


def _diag_body(n1_ref, n2_ref, n1b_ref, n2b_ref, d11_ref, d22_ref, d12_ref):
    r = pl.program_id(0)
    a = n1_ref[...]
    b = n2_ref[...]
    n1b_ref[...] = a.astype(bf16)
    n2b_ref[...] = b.astype(bf16)
    sl = pl.ds(r * RB, RB)
    d11_ref[:, sl] = jnp.sum(a * a, axis=1)[None, :]
    d22_ref[:, sl] = jnp.sum(b * b, axis=1)[None, :]
    d12_ref[:, sl] = jnp.sum(a * b, axis=1)[None, :]


def _tc_diag(n1, n2):
    vec = jax.ShapeDtypeStruct((1, NPAD), f32)
    nb = jax.ShapeDtypeStruct((NPAD, HID), bf16)
    return pl.pallas_call(
        _diag_body,
        grid=(NBLK,),
        in_specs=[pl.BlockSpec((RB, HID), lambda r: (r, 0))] * 2,
        out_specs=[pl.BlockSpec((RB, HID), lambda r: (r, 0))] * 2 + [
            pl.BlockSpec((1, NPAD), lambda r: (0, 0))] * 3,
        out_shape=[nb, nb, vec, vec, vec],
    )(n1, n2)


BL = 512
NLB = NPAD // BL


def _loss_body(n1i_ref, n2i_ref, n1j_ref, n2j_ref,
               d11_ref, d22_ref, d12_ref, out_ref,
               r11, r12, r22, c12):
    i = pl.program_id(0)
    j = pl.program_id(1)
    slj = pl.ds(j * BL, BL)

    a1 = n1i_ref[...]
    a2 = n2i_ref[...]
    b1 = n1j_ref[...]
    b2 = n2j_ref[...]
    dims = (((1,), (1,)), ((), ()))
    s11 = lax.dot_general(a1, b1, dims, preferred_element_type=f32)
    s22 = lax.dot_general(a2, b2, dims, preferred_element_type=f32)
    s12 = lax.dot_general(a1, b2, dims, preferred_element_type=f32)
    s21 = lax.dot_general(a2, b1, dims, preferred_element_type=f32)

    imask = (i * BL + lax.broadcasted_iota(jnp.int32, (BL, 1), 0)) < N
    e11 = jnp.where(imask, jnp.exp(s11 * INV_TAU), 0.0)
    e22 = jnp.where(imask, jnp.exp(s22 * INV_TAU), 0.0)
    e12 = jnp.where(imask, jnp.exp(s12 * INV_TAU), 0.0)
    e21 = jnp.where(imask, jnp.exp(s21 * INV_TAU), 0.0)

    cs11 = jnp.sum(e11, axis=0, keepdims=True)
    cs22 = jnp.sum(e22, axis=0, keepdims=True)
    cs12 = jnp.sum(e12, axis=0, keepdims=True)
    cs21 = jnp.sum(e21, axis=0, keepdims=True)

    @pl.when(i == 0)
    def _():
        r11[:, slj] = cs11
        r22[:, slj] = cs22
        c12[:, slj] = cs12
        r12[:, slj] = cs21

    @pl.when(i > 0)
    def _():
        r11[:, slj] += cs11
        r22[:, slj] += cs22
        c12[:, slj] += cs12
        r12[:, slj] += cs21

    @pl.when((i == NLB - 1) & (j == NLB - 1))
    def _():
        mask = lax.broadcasted_iota(jnp.int32, (1, NPAD), 1) < N
        ed11 = jnp.exp(d11_ref[...] * INV_TAU)
        ed22 = jnp.exp(d22_ref[...] * INV_TAU)
        ld12 = d12_ref[...] * INV_TAU
        l1 = -(ld12 - jnp.log(r11[...] - ed11 + r12[...]))
        l2 = -(ld12 - jnp.log(r22[...] - ed22 + c12[...]))
        tot = jnp.sum(jnp.where(mask, 0.5 * (l1 + l2), 0.0))
        out_ref[...] = (tot / N).reshape(1, 1)


def _tc_loss(n1b, n2b, d11, d22, d12):
    vec = pltpu.VMEM((1, NPAD), f32)
    return pl.pallas_call(
        _loss_body,
        grid=(NLB, NLB),
        in_specs=[
            pl.BlockSpec((BL, HID), lambda i, j: (i, 0)),
            pl.BlockSpec((BL, HID), lambda i, j: (i, 0)),
            pl.BlockSpec((BL, HID), lambda i, j: (j, 0)),
            pl.BlockSpec((BL, HID), lambda i, j: (j, 0)),
            pl.BlockSpec((1, NPAD), lambda i, j: (0, 0)),
            pl.BlockSpec((1, NPAD), lambda i, j: (0, 0)),
            pl.BlockSpec((1, NPAD), lambda i, j: (0, 0)),
        ],
        out_specs=pl.BlockSpec((1, 1), lambda i, j: (0, 0)),
        out_shape=jax.ShapeDtypeStruct((1, 1), f32),
        scratch_shapes=[vec] * 4,
    )(n1b, n2b, n1b, n2b, d11, d22, d12)


_EPT = EPAD // (NC * NS)
_DCH = NPAD // NS


def _sc_deg(src_p):
    mesh = plsc.VectorSubcoreMesh(core_axis_name="c", subcore_axis_name="s")

    @functools.partial(
        pl.kernel,
        out_type=jax.ShapeDtypeStruct((NC, NPAD), f32),
        mesh=mesh,
        scratch_types=[
            pltpu.VMEM((_EPT,), jnp.int32),
            pltpu.VMEM((NPAD,), f32),
            pltpu.VMEM((NS, _DCH), f32),
            pltpu.VMEM_SHARED((NS, NPAD), f32),
        ],
        compiler_params=pltpu.CompilerParams(needs_layout_passes=False),
    )
    def k(src_hbm, out_hbm, idx_v, acc_v, red_v, shared):
        cid = lax.axis_index("c")
        sid = lax.axis_index("s")
        wid = cid * NS + sid

        zeros16 = jnp.zeros((L,), f32)

        @pl.loop(0, NPAD // L)
        def _(t):
            acc_v[pl.ds(t * L, L)] = zeros16

        pltpu.sync_copy(src_hbm.at[pl.ds(wid * _EPT, _EPT)], idx_v)
        ones16 = jnp.ones((L,), f32)

        @pl.loop(0, _EPT // L)
        def _(t):
            idx = idx_v[pl.ds(t * L, L)]
            plsc.addupdate_scatter(acc_v, [idx], ones16)

        pltpu.sync_copy(acc_v, shared.at[sid])
        plsc.subcore_barrier()
        pltpu.sync_copy(shared.at[:, pl.ds(sid * _DCH, _DCH)], red_v)

        @pl.loop(0, _DCH // L)
        def _(t):
            v = red_v[0, pl.ds(t * L, L)]
            for r in range(1, NS):
                v = v + red_v[r, pl.ds(t * L, L)]
            acc_v[pl.ds(t * L, L)] = v

        pltpu.sync_copy(acc_v.at[pl.ds(0, _DCH)],
                        out_hbm.at[cid, pl.ds(sid * _DCH, _DCH)])

    return k(src_p)


_SCH = 128
_NCH = EPAD // NS // _SCH
_IB = _NCH // 2
_STR = NPAD // NS


def _sc_segsum(hlo, hhi, src3, dst3):
    mesh = plsc.VectorSubcoreMesh(core_axis_name="c", subcore_axis_name="s")
    half = jax.ShapeDtypeStruct((NPAD, HHALF), f32)

    @functools.partial(
        pl.kernel,
        out_type=(half, half),
        mesh=mesh,
        scratch_types=[
            pltpu.VMEM((_IB, _SCH), jnp.int32),
            pltpu.VMEM((_IB, _SCH), jnp.int32),
            pltpu.VMEM((_SCH, HHALF), f32),
            pltpu.VMEM((_SCH, HHALF), f32),
            pltpu.VMEM_SHARED((NPAD, HHALF), f32),
            pltpu.SemaphoreType.DMA,
            pltpu.SemaphoreType.DMA,
        ],
        compiler_params=pltpu.CompilerParams(needs_layout_passes=False),
    )
    def k(hlo_hbm, hhi_hbm, src_hbm, dst_hbm, olo_hbm, ohi_hbm,
          src_v, dst_v, buf0, buf1, acc, sem0, sem1):
        cid = lax.axis_index("c")
        sid = lax.axis_index("s")
        base = sid * _STR

        zeros16 = jnp.zeros((L,), f32)

        @pl.loop(0, _SCH)
        def _(r):
            for cc in range(HHALF // L):
                buf0[r, pl.ds(cc * L, L)] = zeros16

        for off in range(0, _STR, _SCH):
            pltpu.sync_copy(buf0, acc.at[pl.ds(base + off, _SCH)])
        plsc.subcore_barrier()

        def run_half(h_hbm, o_hbm):
            for blk in range(2):
                pltpu.sync_copy(
                    src_hbm.at[sid, pl.ds(blk * _IB, _IB)], src_v)
                pltpu.sync_copy(
                    dst_hbm.at[sid, pl.ds(blk * _IB, _IB)], dst_v)

                pltpu.async_copy(h_hbm.at[src_v.at[0]], buf0, sem0)
                pltpu.async_copy(h_hbm.at[src_v.at[1]], buf1, sem1)

                @pl.loop(0, _IB, step=2)
                def _(ch):
                    pltpu.make_async_copy(h_hbm.at[src_v.at[ch]], buf0,
                                          sem0).wait()
                    pltpu.sync_copy(buf0, acc.at[dst_v.at[ch]], add=True)

                    @pl.when(ch + 2 < _IB)
                    def _():
                        pltpu.async_copy(h_hbm.at[src_v.at[ch + 2]],
                                         buf0, sem0)

                    pltpu.make_async_copy(h_hbm.at[src_v.at[ch + 1]], buf1,
                                          sem1).wait()
                    pltpu.sync_copy(buf1, acc.at[dst_v.at[ch + 1]], add=True)

                    @pl.when(ch + 3 < _IB)
                    def _():
                        pltpu.async_copy(h_hbm.at[src_v.at[ch + 3]],
                                         buf1, sem1)

            plsc.subcore_barrier()
            pltpu.sync_copy(acc.at[pl.ds(base, _STR)],
                            o_hbm.at[pl.ds(base, _STR)])

        @pl.when(cid == 0)
        def _():
            run_half(hlo_hbm, olo_hbm)

        @pl.when(cid == 1)
        def _():
            run_half(hhi_hbm, ohi_hbm)

    return k(hlo, hhi, src3, dst3)


def kernel(x, edge_index_1, edge_index_2, W1, a1, W2, a2,
           fc1_w, fc1_b, fc2_w, fc2_b):
    xp = jnp.zeros((NPAD, D), f32).at[:N].set(x)
    u1 = jax.random.uniform(jax.random.key(7), (1, D), dtype=f32)

    def prep(ei):
        ei = ei.astype(jnp.int32)
        pad = jnp.full((EPAD - E,), PAD_IDX, jnp.int32)
        shape = (NS, _NCH, _SCH)
        src = jnp.concatenate([ei[1], pad]).reshape(shape)
        dst = jnp.concatenate([ei[0], pad]).reshape(shape)
        return src, dst

    src1, dst1 = prep(edge_index_1)
    src2, dst2 = prep(edge_index_2)

    deg2 = _sc_deg(src1.reshape(EPAD))
    keep = _tc_keep_mask(xp, deg2, u1)
    h1lo, h1hi, h2lo, h2hi = _tc_mm1(xp, keep, W1)

    g1lo, g1hi = _sc_segsum(h1lo, h1hi, src1, dst1)
    g2lo, g2hi = _sc_segsum(h2lo, h2hi, src2, dst2)

    a1r = a1.reshape(1, 1)
    t1lo, t1hi = _tc_mm2(g1lo, g1hi, a1r, W2)
    t2lo, t2hi = _tc_mm2(g2lo, g2hi, a1r, W2)

    z1lo, z1hi = _sc_segsum(t1lo, t1hi, src1, dst1)
    z2lo, z2hi = _sc_segsum(t2lo, t2hi, src2, dst2)

    a2r = a2.reshape(1, 1)
    n1 = _tc_proj(z1lo, z1hi, a2r, fc1_w, fc1_b.reshape(1, PROJ),
                  fc2_w, fc2_b.reshape(1, HID))
    n2 = _tc_proj(z2lo, z2hi, a2r, fc1_w, fc1_b.reshape(1, PROJ),
                  fc2_w, fc2_b.reshape(1, HID))

    n1b, n2b, d11, d22, d12 = _tc_diag(n1, n2)
    return _tc_loss(n1b, n2b, d11, d22, d12)[0, 0]
